# baseline (speedup 1.0000x reference)
import jax
import jax.numpy as jnp
from jax.experimental import pallas as pl
from jax.experimental.pallas import tpu as pltpu
from jax.experimental.pallas import tpu_sc as plsc

_EPS = 1e-12

_TN = 1024
_TCB = 2048
_GW = 128


def _dist_argmax_body(x_ref, e_ref, idx_ref, maxval_ref, runidx_ref):
    j = pl.program_id(1)
    nj = pl.num_programs(1)
    x = x_ref[...]
    xn = (x / jnp.maximum(
        jnp.sqrt(jnp.sum(x * x, axis=1, keepdims=True)), _EPS)
          ).astype(jnp.bfloat16)
    e = e_ref[...]
    en = e / jnp.maximum(
        jnp.sqrt(jnp.sum(e * e, axis=1, keepdims=True)), _EPS)
    hi = en.astype(jnp.bfloat16)
    r1 = en - hi.astype(jnp.float32)
    mid = r1.astype(jnp.bfloat16)
    lo = (r1 - mid.astype(jnp.float32)).astype(jnp.bfloat16)
    dn = (((1,), (1,)), ((), ()))
    dist = jax.lax.dot_general(
        xn, lo, dn, preferred_element_type=jnp.float32)
    dist = dist + jax.lax.dot_general(
        xn, mid, dn, preferred_element_type=jnp.float32)
    dist = dist + jax.lax.dot_general(
        xn, hi, dn, preferred_element_type=jnp.float32)
    lmax = jnp.max(dist, axis=1)
    cols = jax.lax.broadcasted_iota(jnp.int32, dist.shape, 1)
    lidx = jnp.min(
        jnp.where(dist == lmax[:, None], cols, _TCB), axis=1) + j * _TCB

    @pl.when(j == 0)
    def _():
        maxval_ref[...] = lmax
        runidx_ref[...] = lidx

    @pl.when(j > 0)
    def _():
        better = lmax > maxval_ref[...]
        maxval_ref[...] = jnp.where(better, lmax, maxval_ref[...])
        runidx_ref[...] = jnp.where(better, lidx, runidx_ref[...])

    @pl.when(j == nj - 1)
    def _():
        idx_ref[0, 0, :] = runidx_ref[...]


def _argmax_tc(xf, e):
    n, d = xf.shape
    c = e.shape[0]
    nt, nc = n // _TN, c // _TCB
    idx3 = pl.pallas_call(
        _dist_argmax_body,
        grid=(nt, nc),
        in_specs=[
            pl.BlockSpec((_TN, d), lambda i, j: (i, 0)),
            pl.BlockSpec((_TCB, d), lambda i, j: (j, 0)),
        ],
        out_specs=pl.BlockSpec((1, 1, _TN), lambda i, j: (i, 0, 0)),
        out_shape=jax.ShapeDtypeStruct((nt, 1, _TN), jnp.int32),
        scratch_shapes=[
            pltpu.VMEM((_TN,), jnp.float32),
            pltpu.VMEM((_TN,), jnp.int32),
        ],
        compiler_params=pltpu.CompilerParams(
            dimension_semantics=("parallel", "arbitrary")),
    )(xf, e)
    return idx3.reshape(n)


def _gather_sc(e, idx):
    n = idx.shape[0]
    d = e.shape[1]
    idx2 = idx.reshape(1, n)

    @pl.kernel(
        out_type=jax.ShapeDtypeStruct((n, d), e.dtype),
        mesh=plsc.VectorSubcoreMesh(
            core_axis_name="core", subcore_axis_name="subcore"),
    )
    def k(e_hbm, i_hbm, o_hbm):
        def body(i_vmem, o_vmem):
            pltpu.sync_copy(e_hbm.at[i_vmem.at[0]], o_vmem)

        pltpu.emit_pipeline(
            body,
            grid=(n // _GW,),
            in_specs=[pl.BlockSpec((1, _GW), index_map=lambda i: (0, i))],
            out_specs=[pl.BlockSpec((_GW, d), index_map=lambda i: (i, 0))],
            core_axis_name=("core", "subcore"),
            dimension_semantics=(pltpu.PARALLEL,),
        )(i_hbm, o_hbm)

    return k(e, idx2)


def kernel(x, embed):
    b, nn, d = x.shape
    xf = x.reshape(b * nn, d)
    e = embed[0]
    idx = _argmax_tc(xf, e)
    quantize = _gather_sc(e, idx)
    return quantize.reshape(b, nn, d), idx.reshape(b, nn)
